# baseline (speedup 1.0000x reference)
import functools

import jax
import jax.numpy as jnp
from jax import lax
from jax.experimental import pallas as pl
from jax.experimental.pallas import tpu as pltpu
from jax.experimental.pallas import tpu_sc as plsc

VOCAB = 1000000
DIM = 64
N = 16384

NC = 2
NS = 16
NW = NC * NS
B_W = N // NW
CHUNK = 128
NCHUNK = B_W // CHUNK

W_BLK = 2048
NB = -(-VOCAB // W_BLK)
CPAD = NB * W_BLK
SLICE = CPAD // NS


@functools.partial(
    pl.kernel,
    mesh=plsc.VectorSubcoreMesh(core_axis_name="c", subcore_axis_name="s"),
    out_type=jax.ShapeDtypeStruct((NC, CPAD), jnp.float32),
    scratch_types=[
        pltpu.VMEM((NCHUNK, CHUNK), jnp.int32),
        pltpu.VMEM((CHUNK,), jnp.float32),
        pltpu.VMEM_SHARED((CPAD,), jnp.float32),
    ],
)
def _histogram(idx_hbm, zeros_hbm, out_hbm, idx_v, ones_v, c_shared):
    cid = lax.axis_index("c")
    sid = lax.axis_index("s")
    wid = sid * NC + cid
    pltpu.sync_copy(idx_hbm.at[wid], idx_v)
    for k in range(CHUNK // 16):
        ones_v[pl.ds(k * 16, 16)] = jnp.ones((16,), jnp.float32)
    pltpu.sync_copy(zeros_hbm.at[pl.ds(sid * SLICE, SLICE)],
                    c_shared.at[pl.ds(sid * SLICE, SLICE)])
    plsc.subcore_barrier()
    for j in range(NCHUNK):
        pltpu.sync_copy(ones_v, c_shared.at[idx_v.at[j]], add=True)
    plsc.subcore_barrier()
    pltpu.sync_copy(c_shared.at[pl.ds(sid * SLICE, SLICE)],
                    out_hbm.at[cid, pl.ds(sid * SLICE, SLICE)])


def _matvec_body(t_ref, c_ref, w_ref, o_ref, acc_ref):
    b = pl.program_id(0)

    @pl.when(b == 0)
    def _():
        acc_ref[...] = jnp.zeros_like(acc_ref)

    cb = c_ref[0, :] + c_ref[1, :]
    acc_ref[...] += jax.lax.dot_general(
        t_ref[...], cb[:, None], (((1,), (0,)), ((), ())),
        preferred_element_type=jnp.float32,
        precision=jax.lax.Precision.HIGHEST)

    @pl.when(b == NB - 1)
    def _():
        o_ref[...] = jnp.sum(acc_ref[...] * w_ref[...]).reshape(1, 1) * (1.0 / N)


_matvec = pl.pallas_call(
    _matvec_body,
    grid=(NB,),
    in_specs=[
        pl.BlockSpec((DIM, W_BLK), lambda b: (0, b)),
        pl.BlockSpec((NC, W_BLK), lambda b: (0, b)),
        pl.BlockSpec((DIM, 1), lambda b: (0, 0)),
    ],
    out_specs=pl.BlockSpec((1, 1), lambda b: (0, 0)),
    out_shape=jax.ShapeDtypeStruct((1, 1), jnp.float32),
    scratch_shapes=[pltpu.VMEM((DIM, 1), jnp.float32)],
)


def kernel(word_idxs, table, weights):
    idx = word_idxs.astype(jnp.int32).reshape(NW, NCHUNK, CHUNK)
    zeros = jnp.zeros((CPAD,), jnp.float32)
    counts = _histogram(idx, zeros)
    prod = _matvec(table.T, counts, weights.reshape(DIM, 1))
    return jnp.reshape(prod, ())


# speedup vs baseline: 2.3623x; 2.3623x over previous
import functools

import jax
import jax.numpy as jnp
from jax import lax
from jax.experimental import pallas as pl
from jax.experimental.pallas import tpu as pltpu
from jax.experimental.pallas import tpu_sc as plsc

VOCAB = 1000000
DIM = 64
N = 16384

NC = 2
NS = 16
NW = NC * NS
B_W = N // NW
CHUNK = 128
NCHUNK = B_W // CHUNK

W_BLK = 4096
NB = -(-VOCAB // W_BLK)
CPAD = NB * W_BLK
SLICE = CPAD // NS


@functools.partial(
    pl.kernel,
    mesh=plsc.VectorSubcoreMesh(core_axis_name="c", subcore_axis_name="s"),
    out_type=jax.ShapeDtypeStruct((NC, CPAD), jnp.float32),
    scratch_types=[
        pltpu.VMEM((NCHUNK, CHUNK), jnp.int32),
        pltpu.VMEM((CHUNK,), jnp.float32),
        pltpu.VMEM_SHARED((CPAD,), jnp.float32),
    ],
)
def _histogram(idx_hbm, zeros_hbm, out_hbm, idx_v, ones_v, c_shared):
    cid = lax.axis_index("c")
    sid = lax.axis_index("s")
    wid = sid * NC + cid
    pltpu.sync_copy(idx_hbm.at[wid], idx_v)
    for k in range(CHUNK // 16):
        ones_v[pl.ds(k * 16, 16)] = jnp.ones((16,), jnp.float32)
    pltpu.sync_copy(zeros_hbm.at[pl.ds(sid * SLICE, SLICE)],
                    c_shared.at[pl.ds(sid * SLICE, SLICE)])
    plsc.subcore_barrier()
    for j in range(NCHUNK):
        pltpu.sync_copy(ones_v, c_shared.at[idx_v.at[j]], add=True)
    plsc.subcore_barrier()
    pltpu.sync_copy(c_shared.at[pl.ds(sid * SLICE, SLICE)],
                    out_hbm.at[cid, pl.ds(sid * SLICE, SLICE)])


def _matvec_body(t_ref, c_ref, w_ref, o_ref, acc_ref):
    b = pl.program_id(0)

    @pl.when(b == 0)
    def _():
        acc_ref[...] = jnp.zeros_like(acc_ref)

    s = None
    for g in range(W_BLK // 128):
        sl = pl.ds(g * 128, 128)
        cb = c_ref[0, sl] + c_ref[1, sl]
        part = t_ref[:, sl] * cb[None, :]
        s = part if s is None else s + part
    acc_ref[...] += s

    @pl.when(b == NB - 1)
    def _():
        h = jnp.sum(acc_ref[...], axis=1, keepdims=True)
        o_ref[...] = jnp.sum(h * w_ref[...]).reshape(1, 1) * (1.0 / N)


_matvec = pl.pallas_call(
    _matvec_body,
    grid=(NB,),
    in_specs=[
        pl.BlockSpec((DIM, W_BLK), lambda b: (0, b)),
        pl.BlockSpec((NC, W_BLK), lambda b: (0, b)),
        pl.BlockSpec((DIM, 1), lambda b: (0, 0)),
    ],
    out_specs=pl.BlockSpec((1, 1), lambda b: (0, 0)),
    out_shape=jax.ShapeDtypeStruct((1, 1), jnp.float32),
    scratch_shapes=[pltpu.VMEM((DIM, 128), jnp.float32)],
)


def kernel(word_idxs, table, weights):
    idx = word_idxs.astype(jnp.int32).reshape(NW, NCHUNK, CHUNK)
    zeros = jnp.zeros((CPAD,), jnp.float32)
    counts = _histogram(idx, zeros)
    prod = _matvec(table.T, counts, weights.reshape(DIM, 1))
    return jnp.reshape(prod, ())


# speedup vs baseline: 3.2868x; 1.3914x over previous
import functools

import jax
import jax.numpy as jnp
from jax import lax
from jax.experimental import pallas as pl
from jax.experimental.pallas import tpu as pltpu
from jax.experimental.pallas import tpu_sc as plsc

VOCAB = 1000000
DIM = 64
N = 16384

NC = 2
NS = 16
NW = NC * NS
B_W = N // NW
CHUNK = 128
NCHUNK = B_W // CHUNK

W_BLK = 8192
NB = -(-VOCAB // W_BLK)
CPAD = NB * W_BLK
SLICE = CPAD // NS


@functools.partial(
    pl.kernel,
    mesh=plsc.VectorSubcoreMesh(core_axis_name="c", subcore_axis_name="s"),
    out_type=jax.ShapeDtypeStruct((NC, CPAD), jnp.float32),
    scratch_types=[
        pltpu.VMEM((NCHUNK, CHUNK), jnp.int32),
        pltpu.VMEM((CHUNK,), jnp.float32),
        pltpu.VMEM_SHARED((CPAD,), jnp.float32),
    ],
)
def _histogram(idx_hbm, zeros_hbm, out_hbm, idx_v, ones_v, c_shared):
    cid = lax.axis_index("c")
    sid = lax.axis_index("s")
    wid = sid * NC + cid
    pltpu.sync_copy(idx_hbm.at[wid], idx_v)
    for k in range(CHUNK // 16):
        ones_v[pl.ds(k * 16, 16)] = jnp.ones((16,), jnp.float32)
    pltpu.sync_copy(zeros_hbm.at[pl.ds(sid * SLICE, SLICE)],
                    c_shared.at[pl.ds(sid * SLICE, SLICE)])
    plsc.subcore_barrier()
    for j in range(NCHUNK):
        pltpu.sync_copy(ones_v, c_shared.at[idx_v.at[j]], add=True)
    plsc.subcore_barrier()
    pltpu.sync_copy(c_shared.at[pl.ds(sid * SLICE, SLICE)],
                    out_hbm.at[cid, pl.ds(sid * SLICE, SLICE)])


def _matvec_body(t_ref, c_ref, w_ref, o_ref, acc_ref):
    b = pl.program_id(0)

    @pl.when(b == 0)
    def _():
        acc_ref[...] = jnp.zeros_like(acc_ref)

    s = None
    for g in range(W_BLK // 128):
        sl = pl.ds(g * 128, 128)
        cb = c_ref[0, sl] + c_ref[1, sl]
        part = t_ref[:, sl] * cb[None, :]
        s = part if s is None else s + part
    acc_ref[...] += s

    @pl.when(b == NB - 1)
    def _():
        h = jnp.sum(acc_ref[...], axis=1, keepdims=True)
        o_ref[...] = jnp.sum(h * w_ref[...]).reshape(1, 1) * (1.0 / N)


_matvec = pl.pallas_call(
    _matvec_body,
    grid=(NB,),
    in_specs=[
        pl.BlockSpec((DIM, W_BLK), lambda b: (0, b)),
        pl.BlockSpec((NC, W_BLK), lambda b: (0, b)),
        pl.BlockSpec((DIM, 1), lambda b: (0, 0)),
    ],
    out_specs=pl.BlockSpec((1, 1), lambda b: (0, 0)),
    out_shape=jax.ShapeDtypeStruct((1, 1), jnp.float32),
    scratch_shapes=[pltpu.VMEM((DIM, 128), jnp.float32)],
)


def kernel(word_idxs, table, weights):
    idx = word_idxs.astype(jnp.int32).reshape(NW, NCHUNK, CHUNK)
    zeros = jnp.zeros((CPAD,), jnp.float32)
    counts = _histogram(idx, zeros)
    prod = _matvec(table.T, counts, weights.reshape(DIM, 1))
    return jnp.reshape(prod, ())


# speedup vs baseline: 4.1451x; 1.2611x over previous
import functools

import jax
import jax.numpy as jnp
from jax import lax
from jax.experimental import pallas as pl
from jax.experimental.pallas import tpu as pltpu
from jax.experimental.pallas import tpu_sc as plsc

VOCAB = 1000000
DIM = 64
N = 16384

NC = 2
NS = 16
NW = NC * NS
B_W = N // NW
CHUNK = 128
NCHUNK = B_W // CHUNK

W_BLK = 16384
NB = -(-VOCAB // W_BLK)
CPAD = NB * W_BLK
SLICE = CPAD // NS


@functools.partial(
    pl.kernel,
    mesh=plsc.VectorSubcoreMesh(core_axis_name="c", subcore_axis_name="s"),
    out_type=jax.ShapeDtypeStruct((NC, CPAD), jnp.float32),
    scratch_types=[
        pltpu.VMEM((NCHUNK, CHUNK), jnp.int32),
        pltpu.VMEM((CHUNK,), jnp.float32),
        pltpu.VMEM_SHARED((CPAD,), jnp.float32),
    ],
)
def _histogram(idx_hbm, zeros_hbm, out_hbm, idx_v, ones_v, c_shared):
    cid = lax.axis_index("c")
    sid = lax.axis_index("s")
    wid = sid * NC + cid
    pltpu.sync_copy(idx_hbm.at[wid], idx_v)
    for k in range(CHUNK // 16):
        ones_v[pl.ds(k * 16, 16)] = jnp.ones((16,), jnp.float32)
    pltpu.sync_copy(zeros_hbm.at[pl.ds(sid * SLICE, SLICE)],
                    c_shared.at[pl.ds(sid * SLICE, SLICE)])
    plsc.subcore_barrier()
    for j in range(NCHUNK):
        pltpu.sync_copy(ones_v, c_shared.at[idx_v.at[j]], add=True)
    plsc.subcore_barrier()
    pltpu.sync_copy(c_shared.at[pl.ds(sid * SLICE, SLICE)],
                    out_hbm.at[cid, pl.ds(sid * SLICE, SLICE)])


def _matvec_body(t_ref, c_ref, w_ref, o_ref, acc_ref):
    b = pl.program_id(0)

    @pl.when(b == 0)
    def _():
        acc_ref[...] = jnp.zeros_like(acc_ref)

    s = None
    for g in range(W_BLK // 128):
        sl = pl.ds(g * 128, 128)
        cb = c_ref[0, sl] + c_ref[1, sl]
        part = t_ref[:, sl] * cb[None, :]
        s = part if s is None else s + part
    acc_ref[...] += s

    @pl.when(b == NB - 1)
    def _():
        h = jnp.sum(acc_ref[...], axis=1, keepdims=True)
        o_ref[...] = jnp.sum(h * w_ref[...]).reshape(1, 1) * (1.0 / N)


_matvec = pl.pallas_call(
    _matvec_body,
    grid=(NB,),
    in_specs=[
        pl.BlockSpec((DIM, W_BLK), lambda b: (0, b)),
        pl.BlockSpec((NC, W_BLK), lambda b: (0, b)),
        pl.BlockSpec((DIM, 1), lambda b: (0, 0)),
    ],
    out_specs=pl.BlockSpec((1, 1), lambda b: (0, 0)),
    out_shape=jax.ShapeDtypeStruct((1, 1), jnp.float32),
    scratch_shapes=[pltpu.VMEM((DIM, 128), jnp.float32)],
)


def kernel(word_idxs, table, weights):
    idx = word_idxs.astype(jnp.int32).reshape(NW, NCHUNK, CHUNK)
    zeros = jnp.zeros((CPAD,), jnp.float32)
    counts = _histogram(idx, zeros)
    prod = _matvec(table.T, counts, weights.reshape(DIM, 1))
    return jnp.reshape(prod, ())


# speedup vs baseline: 4.5398x; 1.0952x over previous
import functools

import jax
import jax.numpy as jnp
from jax import lax
from jax.experimental import pallas as pl
from jax.experimental.pallas import tpu as pltpu
from jax.experimental.pallas import tpu_sc as plsc

VOCAB = 1000000
DIM = 64
N = 16384

NC = 2
NS = 16
NW = NC * NS
B_W = N // NW
CHUNK = 128
NCHUNK = B_W // CHUNK

W_BLK = 32768
NB = -(-VOCAB // W_BLK)
CPAD = NB * W_BLK
SLICE = CPAD // NS


@functools.partial(
    pl.kernel,
    mesh=plsc.VectorSubcoreMesh(core_axis_name="c", subcore_axis_name="s"),
    out_type=jax.ShapeDtypeStruct((NC, CPAD), jnp.float32),
    scratch_types=[
        pltpu.VMEM((NCHUNK, CHUNK), jnp.int32),
        pltpu.VMEM((CHUNK,), jnp.float32),
        pltpu.VMEM_SHARED((CPAD,), jnp.float32),
    ],
)
def _histogram(idx_hbm, zeros_hbm, out_hbm, idx_v, ones_v, c_shared):
    cid = lax.axis_index("c")
    sid = lax.axis_index("s")
    wid = sid * NC + cid
    pltpu.sync_copy(idx_hbm.at[wid], idx_v)
    for k in range(CHUNK // 16):
        ones_v[pl.ds(k * 16, 16)] = jnp.ones((16,), jnp.float32)
    pltpu.sync_copy(zeros_hbm.at[pl.ds(sid * SLICE, SLICE)],
                    c_shared.at[pl.ds(sid * SLICE, SLICE)])
    plsc.subcore_barrier()
    for j in range(NCHUNK):
        pltpu.sync_copy(ones_v, c_shared.at[idx_v.at[j]], add=True)
    plsc.subcore_barrier()
    pltpu.sync_copy(c_shared.at[pl.ds(sid * SLICE, SLICE)],
                    out_hbm.at[cid, pl.ds(sid * SLICE, SLICE)])


def _matvec_body(t_ref, c_ref, w_ref, o_ref, acc_ref):
    b = pl.program_id(0)

    @pl.when(b == 0)
    def _():
        acc_ref[...] = jnp.zeros_like(acc_ref)

    s = None
    for g in range(W_BLK // 128):
        sl = pl.ds(g * 128, 128)
        cb = c_ref[0, sl] + c_ref[1, sl]
        part = t_ref[:, sl] * cb[None, :]
        s = part if s is None else s + part
    acc_ref[...] += s

    @pl.when(b == NB - 1)
    def _():
        h = jnp.sum(acc_ref[...], axis=1, keepdims=True)
        o_ref[...] = jnp.sum(h * w_ref[...]).reshape(1, 1) * (1.0 / N)


_matvec = pl.pallas_call(
    _matvec_body,
    grid=(NB,),
    in_specs=[
        pl.BlockSpec((DIM, W_BLK), lambda b: (0, b)),
        pl.BlockSpec((NC, W_BLK), lambda b: (0, b)),
        pl.BlockSpec((DIM, 1), lambda b: (0, 0)),
    ],
    out_specs=pl.BlockSpec((1, 1), lambda b: (0, 0)),
    out_shape=jax.ShapeDtypeStruct((1, 1), jnp.float32),
    scratch_shapes=[pltpu.VMEM((DIM, 128), jnp.float32)],
)


def kernel(word_idxs, table, weights):
    idx = word_idxs.astype(jnp.int32).reshape(NW, NCHUNK, CHUNK)
    zeros = jnp.zeros((CPAD,), jnp.float32)
    counts = _histogram(idx, zeros)
    prod = _matvec(table.T, counts, weights.reshape(DIM, 1))
    return jnp.reshape(prod, ())
